# baseline (speedup 1.0000x reference)
import functools

import jax
import jax.numpy as jnp
from jax import lax
from jax.experimental import pallas as pl
from jax.experimental.pallas import tpu as pltpu
from jax.experimental.pallas import tpu_sc as plsc

_NC = 2
_NS = 16
_NW = _NC * _NS

_CHUNK = 128
_GROUP = 4
_NBUF = 3


@functools.partial(jax.jit, static_argnames=("rows_w", "f", "d"))
def _gather_call(x, table, *, rows_w, f, d):
    b_per_w = rows_w * f
    chunks_per_w = b_per_w // _CHUNK
    groups = chunks_per_w // _GROUP
    rpg = _GROUP * _CHUNK
    b_total = b_per_w * _NW
    assert groups >= _NBUF + 1

    mesh = plsc.VectorSubcoreMesh(
        core_axis_name="c", subcore_axis_name="s",
        num_cores=_NC, num_subcores=_NS,
    )

    @functools.partial(
        pl.kernel,
        out_type=jax.ShapeDtypeStruct((b_total, d), jnp.float32),
        mesh=mesh,
        scratch_types=[
            pltpu.VMEM((rows_w, f), jnp.int32),
            pltpu.VMEM((b_per_w,), jnp.int32),
            pltpu.VMEM((_NBUF, rpg, d), jnp.float32),
            pltpu.SemaphoreType.DMA((_NBUF,)),
            pltpu.SemaphoreType.DMA((_NBUF,)),
        ],
        compiler_params=pltpu.CompilerParams(use_tc_tiling_on_sc=False),
    )
    def body(idx_hbm, table_hbm, out_hbm, sx, idxc, rows_v, gsem, wsem):
        wid = lax.axis_index("s") * _NC + lax.axis_index("c")
        base = wid * b_per_w
        pltpu.sync_copy(idx_hbm.at[pl.ds(wid * rows_w, rows_w)], sx)

        def compact(r, _):
            lo = sx[r, pl.ds(0, 16)]
            hi = sx[r, pl.ds(f - 16, 16)]
            idxc[pl.ds(r * f, 16)] = lo
            idxc[pl.ds(r * f + f - 16, 16)] = hi
            return ()

        lax.fori_loop(0, rows_w, compact, (), unroll=4)

        def fire_g(g, b):
            for j in range(_GROUP):
                pltpu.async_copy(
                    table_hbm.at[idxc.at[pl.ds((g * _GROUP + j) * _CHUNK,
                                               _CHUNK)]],
                    rows_v.at[b, pl.ds(j * _CHUNK, _CHUNK)],
                    gsem.at[b])

        def drain_g(b):
            pltpu.make_async_copy(
                table_hbm.at[pl.ds(0, rpg)], rows_v.at[b], gsem.at[b]).wait()

        def fire_w(g, b):
            pltpu.async_copy(rows_v.at[b],
                             out_hbm.at[pl.ds(base + g * rpg, rpg)],
                             wsem.at[b])

        def wait_w(b):
            pltpu.make_async_copy(rows_v.at[b],
                                  out_hbm.at[pl.ds(base, rpg)],
                                  wsem.at[b]).wait()

        fire_g(0, 0)
        fire_g(1, 1)
        drain_g(0)
        fire_w(0, 0)
        fire_g(2, 2)
        drain_g(1)
        fire_w(1, 1)

        def step(g, _):
            b = g % _NBUF
            bn = (g + 1) % _NBUF
            wait_w(bn)
            fire_g(g + 1, bn)
            drain_g(b)
            fire_w(g, b)
            return ()

        lax.fori_loop(2, groups - 1, step, (), unroll=False)

        g = groups - 1
        wait_w((g + 1) % _NBUF)
        drain_g(g % _NBUF)
        fire_w(g, g % _NBUF)
        wait_w((groups - 2) % _NBUF)
        wait_w((groups - 1) % _NBUF)

    return body(x, table)


def kernel(x, table):
    n_rows, f = x.shape
    d = table.shape[1]
    assert 16 <= f <= 32
    b_total = n_rows * f
    assert b_total % (_NW * _CHUNK * _GROUP) == 0
    rows_w = n_rows // _NW
    out = _gather_call(x.astype(jnp.int32), table, rows_w=rows_w, f=f, d=d)
    return out.reshape(n_rows, f, d)
